# baseline (speedup 1.0000x reference)
import functools

import jax
import jax.numpy as jnp
from jax import lax
from jax.experimental import pallas as pl
from jax.experimental.pallas import tpu as pltpu
from jax.experimental.pallas import tpu_sc as plsc

_SIGMA = 0.1
_K = 10
_HALF = 32
_CPS = 32 * 8192
_LANES = 16
_NC = 2
_GV = 64
_GSIZE = _GV * _LANES
_CHUNK = 32768
_ROW = 8192
_RPC = _CHUNK // _ROW
_GPC = _CHUNK // _GSIZE
_NCH = _CPS // _CHUNK
_PAIRS = _NCH // 2


def _tree_max(vs):
    vs = list(vs)
    while len(vs) > 1:
        nxt = [jnp.maximum(vs[i], vs[i + 1]) for i in range(0, len(vs) - 1, 2)]
        if len(vs) % 2:
            nxt.append(vs[-1])
        vs = nxt
    return vs[0]


@functools.partial(
    pl.kernel,
    out_type=jax.ShapeDtypeStruct((_HALF, _K * _LANES), jnp.float32),
    mesh=plsc.VectorSubcoreMesh(core_axis_name="c", subcore_axis_name="s"),
    compiler_params=pltpu.CompilerParams(needs_layout_passes=False),
    scratch_types=[
        pltpu.VMEM((_CHUNK,), jnp.float32),
        pltpu.VMEM((_CHUNK,), jnp.float32),
        pltpu.VMEM((_K * _LANES,), jnp.float32),
        pltpu.SemaphoreType.DMA,
        pltpu.SemaphoreType.DMA,
    ],
)
def _sc_partials(feat_hbm, out_hbm, buf0, buf1, state, sem0, sem1):
    wid = lax.axis_index("s") * _NC + lax.axis_index("c")

    neg = jnp.full((_LANES,), -jnp.inf, jnp.float32)
    for j in range(_K):
        state[pl.ds(j * _LANES, _LANES)] = neg

    def chunk_copy(ci, buf, sem):
        class _Chunk:
            def __init__(self):
                self.copies = [
                    pltpu.make_async_copy(
                        feat_hbm.at[_HALF + wid, _RPC * ci + r, :],
                        buf.at[pl.ds(r * _ROW, _ROW)], sem)
                    for r in range(_RPC)]

            def start(self):
                for c in self.copies:
                    c.start()

            def wait(self):
                for c in self.copies:
                    c.wait()

        return _Chunk()

    def scan_chunk(buf, thr):
        def group_body(gi, thr):
            b = pl.multiple_of(gi * _GSIZE, _GSIZE)
            xs = [buf[pl.ds(b + u * _LANES, _LANES)] for u in range(_GV)]
            gs = jnp.max(_tree_max(xs))

            def insert(thr):
                t = [state[pl.ds(j * _LANES, _LANES)] for j in range(_K)]
                for x in xs:
                    for j in range(_K):
                        hi = jnp.maximum(t[j], x)
                        x = jnp.minimum(t[j], x)
                        t[j] = hi
                for j in range(_K):
                    state[pl.ds(j * _LANES, _LANES)] = t[j]
                return jnp.max(t[_K - 1])

            return lax.cond(gs > thr, insert, lambda s: s, thr)

        return lax.fori_loop(0, _GPC, group_body, thr)

    thr = jnp.float32(-jnp.inf)
    chunk_copy(0, buf0, sem0).start()
    for p in range(_PAIRS):
        chunk_copy(2 * p + 1, buf1, sem1).start()
        chunk_copy(2 * p, buf0, sem0).wait()
        thr = scan_chunk(buf0, thr)
        if p + 1 < _PAIRS:
            chunk_copy(2 * p + 2, buf0, sem0).start()
        chunk_copy(2 * p + 1, buf1, sem1).wait()
        thr = scan_chunk(buf1, thr)

    pltpu.sync_copy(state, out_hbm.at[wid])


def _nor_body(nor_ref, o_ref, acc_ref):
    i = pl.program_id(0)

    @pl.when(i == 0)
    def _():
        acc_ref[0] = jnp.float32(0.0)

    acc_ref[0] += jnp.sum(nor_ref[...])

    @pl.when(i == pl.num_programs(0) - 1)
    def _():
        o_ref[...] = jnp.zeros((1, 1), jnp.float32) + acc_ref[0]


def _finish_body(p_ref, n_ref, o_ref):
    cand = p_ref[...]
    iota = lax.broadcasted_iota(jnp.int32, cand.shape, 1)
    s = jnp.zeros((_HALF, 1), jnp.float32)
    for _ in range(_K):
        m = jnp.max(cand, axis=1, keepdims=True)
        s = s + m
        eq = cand == m
        first = jnp.min(jnp.where(eq, iota, jnp.int32(2**30)), axis=1,
                        keepdims=True)
        cand = jnp.where(eq & (iota == first), -jnp.inf, cand)
    loss_abn = jnp.sum(s) / (_K * _HALF)
    loss_nor = n_ref[0, 0] / (_HALF * _CPS)
    o_ref[...] = jnp.zeros((1, 1), jnp.float32) + (loss_abn - (loss_nor + _SIGMA))


def kernel(features):
    partials = _sc_partials(features)
    norsum = pl.pallas_call(
        _nor_body,
        grid=(_HALF,),
        in_specs=[pl.BlockSpec((1, 32, 8192), lambda i: (i, 0, 0))],
        out_specs=pl.BlockSpec((1, 1), lambda i: (0, 0)),
        out_shape=jax.ShapeDtypeStruct((1, 1), jnp.float32),
        scratch_shapes=[pltpu.SMEM((1,), jnp.float32)],
    )(features)
    out = pl.pallas_call(
        _finish_body,
        out_shape=jax.ShapeDtypeStruct((1, 1), jnp.float32),
    )(partials, norsum)
    return out[0, 0]


# speedup vs baseline: 1.1178x; 1.1178x over previous
import functools

import jax
import jax.numpy as jnp
from jax import lax
from jax.experimental import pallas as pl
from jax.experimental.pallas import tpu as pltpu
from jax.experimental.pallas import tpu_sc as plsc

_SIGMA = 0.1
_K = 10
_HALF = 32
_CPS = 32 * 8192
_LANES = 16
_NC = 2
_GV = 32
_GSIZE = _GV * _LANES
_CHUNK = 32768
_ROW = 8192
_RPC = _CHUNK // _ROW
_GPC = _CHUNK // _GSIZE
_NCH = _CPS // _CHUNK
_PAIRS = _NCH // 2


def _tree_max(vs):
    vs = list(vs)
    while len(vs) > 1:
        nxt = [jnp.maximum(vs[i], vs[i + 1]) for i in range(0, len(vs) - 1, 2)]
        if len(vs) % 2:
            nxt.append(vs[-1])
        vs = nxt
    return vs[0]


@functools.partial(
    pl.kernel,
    out_type=jax.ShapeDtypeStruct((_HALF, _K * _LANES), jnp.float32),
    mesh=plsc.VectorSubcoreMesh(core_axis_name="c", subcore_axis_name="s"),
    compiler_params=pltpu.CompilerParams(needs_layout_passes=False),
    scratch_types=[
        pltpu.VMEM((_CHUNK,), jnp.float32),
        pltpu.VMEM((_CHUNK,), jnp.float32),
        pltpu.VMEM((_K * _LANES,), jnp.float32),
        pltpu.SemaphoreType.DMA,
        pltpu.SemaphoreType.DMA,
    ],
)
def _sc_partials(feat_hbm, out_hbm, buf0, buf1, state, sem0, sem1):
    wid = lax.axis_index("s") * _NC + lax.axis_index("c")

    neg = jnp.full((_LANES,), -jnp.inf, jnp.float32)
    for j in range(_K):
        state[pl.ds(j * _LANES, _LANES)] = neg

    def chunk_copy(ci, buf, sem):
        class _Chunk:
            def __init__(self):
                self.copies = [
                    pltpu.make_async_copy(
                        feat_hbm.at[_HALF + wid, _RPC * ci + r, :],
                        buf.at[pl.ds(r * _ROW, _ROW)], sem)
                    for r in range(_RPC)]

            def start(self):
                for c in self.copies:
                    c.start()

            def wait(self):
                for c in self.copies:
                    c.wait()

        return _Chunk()

    def scan_chunk(buf, thr):
        def group_body(gi, thr):
            b = pl.multiple_of(gi * 2 * _GSIZE, 2 * _GSIZE)
            xss = [
                [buf[pl.ds(b + h * _GSIZE + u * _LANES, _LANES)]
                 for u in range(_GV)]
                for h in range(2)
            ]
            gss = [jnp.max(_tree_max(xs)) for xs in xss]

            def mk_insert(xs):
                def insert(thr):
                    t = [state[pl.ds(j * _LANES, _LANES)] for j in range(_K)]
                    for x in xs:
                        for j in range(_K):
                            hi = jnp.maximum(t[j], x)
                            x = jnp.minimum(t[j], x)
                            t[j] = hi
                    for j in range(_K):
                        state[pl.ds(j * _LANES, _LANES)] = t[j]
                    return jnp.max(t[_K - 1])
                return insert

            for h in range(2):
                thr = lax.cond(gss[h] > thr, mk_insert(xss[h]),
                               lambda s: s, thr)
            return thr

        return lax.fori_loop(0, _GPC // 2, group_body, thr)

    thr = jnp.float32(-jnp.inf)
    chunk_copy(0, buf0, sem0).start()
    for p in range(_PAIRS):
        chunk_copy(2 * p + 1, buf1, sem1).start()
        chunk_copy(2 * p, buf0, sem0).wait()
        thr = scan_chunk(buf0, thr)
        if p + 1 < _PAIRS:
            chunk_copy(2 * p + 2, buf0, sem0).start()
        chunk_copy(2 * p + 1, buf1, sem1).wait()
        thr = scan_chunk(buf1, thr)

    pltpu.sync_copy(state, out_hbm.at[wid])


def _nor_body(nor_ref, o_ref, acc_ref):
    i = pl.program_id(0)

    @pl.when(i == 0)
    def _():
        acc_ref[0] = jnp.float32(0.0)

    acc_ref[0] += jnp.sum(nor_ref[...])

    @pl.when(i == pl.num_programs(0) - 1)
    def _():
        o_ref[...] = jnp.zeros((1, 1), jnp.float32) + acc_ref[0]


def _finish_body(p_ref, n_ref, o_ref):
    cand = p_ref[...]
    iota = lax.broadcasted_iota(jnp.int32, cand.shape, 1)
    s = jnp.zeros((_HALF, 1), jnp.float32)
    for _ in range(_K):
        m = jnp.max(cand, axis=1, keepdims=True)
        s = s + m
        eq = cand == m
        first = jnp.min(jnp.where(eq, iota, jnp.int32(2**30)), axis=1,
                        keepdims=True)
        cand = jnp.where(eq & (iota == first), -jnp.inf, cand)
    loss_abn = jnp.sum(s) / (_K * _HALF)
    loss_nor = n_ref[0, 0] / (_HALF * _CPS)
    o_ref[...] = jnp.zeros((1, 1), jnp.float32) + (loss_abn - (loss_nor + _SIGMA))


def kernel(features):
    partials = _sc_partials(features)
    norsum = pl.pallas_call(
        _nor_body,
        grid=(_HALF,),
        in_specs=[pl.BlockSpec((1, 32, 8192), lambda i: (i, 0, 0))],
        out_specs=pl.BlockSpec((1, 1), lambda i: (0, 0)),
        out_shape=jax.ShapeDtypeStruct((1, 1), jnp.float32),
        scratch_shapes=[pltpu.SMEM((1,), jnp.float32)],
    )(features)
    out = pl.pallas_call(
        _finish_body,
        out_shape=jax.ShapeDtypeStruct((1, 1), jnp.float32),
    )(partials, norsum)
    return out[0, 0]
